# baseline (speedup 1.0000x reference)
import functools

import jax
import jax.numpy as jnp
from jax import lax
from jax.experimental import pallas as pl
from jax.experimental.pallas import tpu as pltpu
from jax.experimental.pallas import tpu_sc as plsc

NUM_USER = 6000
NUM_ITEM = 4000
NTOT = NUM_USER + NUM_ITEM
NPAD = 10240
E = 640000
D = 64
DP = 64

NC = 2
NS = 16
NW = NC * NS
EPW = E // NW
C = 128
NCH = 160
NSCAT = 158
RPT = NPAD // NS
HN = NPAD // NC
JROW = HN
ACCR = HN + 8
OPT = HN // NS

_mesh = plsc.VectorSubcoreMesh(
    core_axis_name="c", subcore_axis_name="s", num_cores=NC, num_subcores=NS)


def _fill_pad(ref, start, count, value, dtype):
    vec = jnp.full((16,), value, dtype)

    def body(g, _):
        ref[pl.ds(start + g * 16, 16)] = vec
        return 0

    lax.fori_loop(0, count // 16, body, 0)


def _load_edges(src_hbm, dst_hbm, slot, srcb, dstb):
    base = slot * EPW
    pltpu.sync_copy(src_hbm.at[pl.ds(base, EPW)], srcb.at[pl.ds(0, EPW)])
    pltpu.sync_copy(dst_hbm.at[pl.ds(base, EPW)], dstb.at[pl.ds(0, EPW)])
    _fill_pad(srcb, EPW, NCH * C - EPW, NTOT, jnp.int32)
    _fill_pad(dstb, EPW, NCH * C - EPW, NTOT, jnp.int32)


def _make_deg():
    @functools.partial(
        pl.kernel,
        out_type=jax.ShapeDtypeStruct((NC, NS, RPT), jnp.float32),
        mesh=_mesh,
        compiler_params=pltpu.CompilerParams(use_tc_tiling_on_sc=False),
        scratch_types=[
            pltpu.VMEM((NCH * C,), jnp.int32),
            pltpu.VMEM((NCH * C,), jnp.int32),
            pltpu.VMEM((NCH, C), jnp.int32),
            pltpu.VMEM((NCH, C), jnp.float32),
            pltpu.VMEM((RPT,), jnp.float32),
            pltpu.VMEM_SHARED((NPAD,), jnp.float32),
            pltpu.SemaphoreType.DMA,
            pltpu.SemaphoreType.DMA,
        ],
    )
    def deg_kernel(src_hbm, dst_hbm, out_hbm, srcb, dstb, src2d, val2d, zdeg,
                   degacc, sem0, sem1):
        cid = lax.axis_index("c")
        sid = lax.axis_index("s")
        wid = sid * NC + cid

        _load_edges(src_hbm, dst_hbm, wid, srcb, dstb)

        def fix(ch, _):
            for k in range(8):
                off = ch * C + k * 16
                s = srcb[pl.ds(off, 16)]
                d = dstb[pl.ds(off, 16)]
                src2d[ch, pl.ds(k * 16, 16)] = s
                val2d[ch, pl.ds(k * 16, 16)] = jnp.where(
                    s == d, jnp.zeros((16,), jnp.float32),
                    jnp.ones((16,), jnp.float32))
            return 0

        lax.fori_loop(0, NCH, fix, 0)

        _fill_pad(zdeg, 0, RPT, 0.0, jnp.float32)
        pltpu.sync_copy(zdeg, degacc.at[pl.ds(sid * RPT, RPT)])
        plsc.subcore_barrier()

        def sstart(j, sem):
            pltpu.async_copy(val2d.at[j], degacc.at[src2d.at[j]], sem, add=True)

        def swait(j, sem):
            pltpu.make_async_copy(val2d.at[j], degacc.at[src2d.at[j]], sem).wait()

        def loop(i, _):
            jj = 2 * i
            sstart(jj, sem0)
            sstart(jj + 1, sem1)
            swait(jj, sem0)
            swait(jj + 1, sem1)
            return 0

        lax.fori_loop(0, NSCAT // 2, loop, 0)
        plsc.subcore_barrier()
        pltpu.sync_copy(degacc.at[pl.ds(sid * RPT, RPT)], out_hbm.at[cid, sid])

    return deg_kernel


NQ = 2
EPQ = E // (NS * NQ)
NBUF = 2
CC = 16
QSCAT = 1250
QCH = QSCAT + NBUF


def _make_conv():
    @functools.partial(
        pl.kernel,
        out_type=jax.ShapeDtypeStruct((NC, NS, OPT, DP), jnp.float32),
        mesh=_mesh,
        compiler_params=pltpu.CompilerParams(use_tc_tiling_on_sc=False),
        scratch_types=[
            pltpu.VMEM((QCH * CC,), jnp.int32),
            pltpu.VMEM((QCH * CC,), jnp.int32),
            pltpu.VMEM((QCH, CC), jnp.int32),
            [pltpu.VMEM((CC, DP), jnp.float32)] * NBUF,
            pltpu.VMEM((64, DP), jnp.float32),
            pltpu.VMEM_SHARED((ACCR, DP), jnp.float32),
            [pltpu.SemaphoreType.DMA] * NBUF,
            [pltpu.SemaphoreType.DMA] * NBUF,
        ],
    )
    def conv_kernel(table_hbm, src_hbm, dst_hbm, out_hbm, srcb, dstb, dst2d,
                    bufs, zbuf, acc, gsems, ssems):
        cid = lax.axis_index("c")
        sid = lax.axis_index("s")
        row0 = cid * HN

        def zfill(r, _):
            for k in range(DP // 16):
                zbuf[r, pl.ds(k * 16, 16)] = jnp.zeros((16,), jnp.float32)
            return 0

        lax.fori_loop(0, 64, zfill, 0)
        for i in range(OPT // 64):
            pltpu.sync_copy(zbuf, acc.at[pl.ds(sid * OPT + i * 64, 64)])

        @pl.when(sid == 0)
        def _():
            pltpu.sync_copy(zbuf.at[pl.ds(0, ACCR - HN)],
                            acc.at[pl.ds(HN, ACCR - HN)])

        def gstart(j, b):
            pltpu.async_copy(
                table_hbm.at[srcb.at[pl.ds(j * CC, CC)]], bufs[b], gsems[b])

        def gwait(j, b):
            pltpu.make_async_copy(
                table_hbm.at[srcb.at[pl.ds(j * CC, CC)]], bufs[b],
                gsems[b]).wait()

        def sstart(j, b):
            pltpu.async_copy(bufs[b], acc.at[dst2d.at[j]], ssems[b], add=True)

        def swait(j, b):
            pltpu.make_async_copy(bufs[b], acc.at[dst2d.at[j]],
                                  ssems[b]).wait()

        for q in range(NQ):
            base = (sid * NQ + q) * EPQ
            pltpu.sync_copy(src_hbm.at[pl.ds(base, EPQ)],
                            srcb.at[pl.ds(0, EPQ)])
            pltpu.sync_copy(dst_hbm.at[pl.ds(base, EPQ)],
                            dstb.at[pl.ds(0, EPQ)])
            _fill_pad(srcb, EPQ, QCH * CC - EPQ, NTOT, jnp.int32)
            _fill_pad(dstb, EPQ, QCH * CC - EPQ, NTOT, jnp.int32)

            def fix(ch, _):
                for k in range(CC // 16):
                    off = ch * CC + k * 16
                    s = srcb[pl.ds(off, 16)]
                    d = dstb[pl.ds(off, 16)]
                    srcb[pl.ds(off, 16)] = jnp.where(
                        s == d, jnp.full((16,), NTOT, jnp.int32), s)
                    dloc = d - row0
                    ooh = (dloc < 0) | (dloc >= HN)
                    dst2d[ch, pl.ds(k * 16, 16)] = jnp.where(
                        ooh, jnp.full((16,), JROW, jnp.int32), dloc)
                return 0

            lax.fori_loop(0, QCH, fix, 0)
            if q == 0:
                plsc.subcore_barrier()

            for b in range(NBUF):
                gstart(b, b)

            def loop(i, _):
                jj = NBUF * i
                for b in range(NBUF):
                    gwait(jj + b, b)
                    sstart(jj + b, b)
                    swait(jj + b, b)
                    gstart(jj + NBUF + b, b)
                return 0

            lax.fori_loop(0, QSCAT // NBUF, loop, 0)
            for b in range(NBUF):
                gwait(QSCAT + b, b)

        plsc.subcore_barrier()
        pltpu.sync_copy(acc.at[pl.ds(sid * OPT, OPT)], out_hbm.at[cid, sid])

    return conv_kernel


def _mlp_body(f_ref, w1_ref, b1_ref, w2_ref, b2_ref, o_ref):
    h = jnp.dot(f_ref[...], w1_ref[...], preferred_element_type=jnp.float32)
    h = h + b1_ref[...]
    h = jnp.where(h > 0, h, 0.01 * h)
    t = jnp.dot(h, w2_ref[...], preferred_element_type=jnp.float32)
    t = t + b2_ref[...]
    n = jnp.sqrt(jnp.sum(t * t, axis=1, keepdims=True))
    o_ref[...] = t / jnp.maximum(n, 1e-12)


def _pref_body(p_ref, o_ref):
    p = p_ref[...]
    n = jnp.sqrt(jnp.sum(p * p, axis=1, keepdims=True))
    o_ref[...] = p / jnp.maximum(n, 1e-12)


def _scale_body(degp_ref, x_ref, dinv_ref, xs_ref):
    deg = degp_ref[:, 0:1] + degp_ref[:, 1:2]
    dinv = jnp.where(deg > 0.0, lax.rsqrt(jnp.maximum(deg, 1e-12)), 0.0)
    dinv_ref[...] = dinv
    xs_ref[...] = x_ref[...] * dinv


def _mid_body(a_ref, dinv_ref, h_ref, ys_ref):
    dinv = dinv_ref[...]
    h = a_ref[...] * dinv
    h_ref[...] = h
    ys_ref[...] = h * dinv


def _final_body(x_ref, h_ref, b_ref, dinv_ref, o_ref):
    o_ref[...] = x_ref[...] + h_ref[...] + b_ref[...] * dinv_ref[...]


_deg_call = _make_deg()
_conv_call = _make_conv()


def kernel(features, preference, W1, b1, W2, b2, edge_index):
    temp_n = pl.pallas_call(
        _mlp_body,
        out_shape=jax.ShapeDtypeStruct((NUM_ITEM, D), jnp.float32),
    )(features, W1, b1.reshape(1, -1), W2, b2.reshape(1, -1))

    pref_n = pl.pallas_call(
        _pref_body,
        out_shape=jax.ShapeDtypeStruct((NUM_USER, D), jnp.float32),
    )(preference)

    x = jnp.concatenate([pref_n, temp_n], axis=0)
    x = jnp.pad(x, ((0, NPAD - NTOT), (0, 0)))

    e_src = edge_index[0]
    e_dst = edge_index[1]

    degp = _deg_call(e_src, e_dst)
    degp_t = degp.reshape(NC, NPAD).T

    dinv, xs = pl.pallas_call(
        _scale_body,
        out_shape=(
            jax.ShapeDtypeStruct((NPAD, 1), jnp.float32),
            jax.ShapeDtypeStruct((NPAD, DP), jnp.float32),
        ),
    )(degp_t, x)

    acc1 = _conv_call(xs, e_src, e_dst).reshape(NPAD, DP)

    h, ys = pl.pallas_call(
        _mid_body,
        out_shape=(
            jax.ShapeDtypeStruct((NPAD, DP), jnp.float32),
            jax.ShapeDtypeStruct((NPAD, DP), jnp.float32),
        ),
    )(acc1, dinv)

    acc2 = _conv_call(ys, e_src, e_dst).reshape(NPAD, DP)

    x_hat = pl.pallas_call(
        _final_body,
        out_shape=jax.ShapeDtypeStruct((NPAD, DP), jnp.float32),
    )(x, h, acc2, dinv)

    return x_hat[:NTOT]


# speedup vs baseline: 1.4259x; 1.4259x over previous
import functools

import jax
import jax.numpy as jnp
from jax import lax
from jax.experimental import pallas as pl
from jax.experimental.pallas import tpu as pltpu
from jax.experimental.pallas import tpu_sc as plsc

NUM_USER = 6000
NUM_ITEM = 4000
NTOT = NUM_USER + NUM_ITEM
NPAD = 10240
E = 640000
D = 64
DP = 64

NC = 2
NS = 16
NW = NC * NS
EPW = E // NW
C = 128
NCH = 160
NSCAT = 158
RPT = NPAD // NS
HN = NPAD // NC
JROW = HN
ACCR = HN + 8
OPT = HN // NS

_mesh = plsc.VectorSubcoreMesh(
    core_axis_name="c", subcore_axis_name="s", num_cores=NC, num_subcores=NS)


def _fill_pad(ref, start, count, value, dtype):
    vec = jnp.full((16,), value, dtype)

    def body(g, _):
        ref[pl.ds(start + g * 16, 16)] = vec
        return 0

    lax.fori_loop(0, count // 16, body, 0)


def _load_edges(src_hbm, dst_hbm, slot, srcb, dstb):
    base = slot * EPW
    pltpu.sync_copy(src_hbm.at[pl.ds(base, EPW)], srcb.at[pl.ds(0, EPW)])
    pltpu.sync_copy(dst_hbm.at[pl.ds(base, EPW)], dstb.at[pl.ds(0, EPW)])
    _fill_pad(srcb, EPW, NCH * C - EPW, NTOT, jnp.int32)
    _fill_pad(dstb, EPW, NCH * C - EPW, NTOT, jnp.int32)


def _make_deg():
    @functools.partial(
        pl.kernel,
        out_type=jax.ShapeDtypeStruct((NC, NS, RPT), jnp.float32),
        mesh=_mesh,
        compiler_params=pltpu.CompilerParams(use_tc_tiling_on_sc=False),
        scratch_types=[
            pltpu.VMEM((NCH * C,), jnp.int32),
            pltpu.VMEM((NCH * C,), jnp.int32),
            pltpu.VMEM((NCH, C), jnp.int32),
            pltpu.VMEM((NCH, C), jnp.float32),
            pltpu.VMEM((RPT,), jnp.float32),
            pltpu.VMEM_SHARED((NPAD,), jnp.float32),
            pltpu.SemaphoreType.DMA,
            pltpu.SemaphoreType.DMA,
        ],
    )
    def deg_kernel(src_hbm, dst_hbm, out_hbm, srcb, dstb, src2d, val2d, zdeg,
                   degacc, sem0, sem1):
        cid = lax.axis_index("c")
        sid = lax.axis_index("s")
        wid = sid * NC + cid

        _load_edges(src_hbm, dst_hbm, wid, srcb, dstb)

        def fix(ch, _):
            for k in range(8):
                off = ch * C + k * 16
                s = srcb[pl.ds(off, 16)]
                d = dstb[pl.ds(off, 16)]
                src2d[ch, pl.ds(k * 16, 16)] = s
                val2d[ch, pl.ds(k * 16, 16)] = jnp.where(
                    s == d, jnp.zeros((16,), jnp.float32),
                    jnp.ones((16,), jnp.float32))
            return 0

        lax.fori_loop(0, NCH, fix, 0)

        _fill_pad(zdeg, 0, RPT, 0.0, jnp.float32)
        pltpu.sync_copy(zdeg, degacc.at[pl.ds(sid * RPT, RPT)])
        plsc.subcore_barrier()

        def sstart(j, sem):
            pltpu.async_copy(val2d.at[j], degacc.at[src2d.at[j]], sem, add=True)

        def swait(j, sem):
            pltpu.make_async_copy(val2d.at[j], degacc.at[src2d.at[j]], sem).wait()

        def loop(i, _):
            jj = 2 * i
            sstart(jj, sem0)
            sstart(jj + 1, sem1)
            swait(jj, sem0)
            swait(jj + 1, sem1)
            return 0

        lax.fori_loop(0, NSCAT // 2, loop, 0)
        plsc.subcore_barrier()
        pltpu.sync_copy(degacc.at[pl.ds(sid * RPT, RPT)], out_hbm.at[cid, sid])

    return deg_kernel


NQ = 2
EPQ = E // (NS * NQ)
NBUF = 2
CC = 32
QSCAT = 626
QCH = QSCAT + NBUF


def _make_conv():
    @functools.partial(
        pl.kernel,
        out_type=jax.ShapeDtypeStruct((NC, NS, OPT, DP), jnp.float32),
        mesh=_mesh,
        compiler_params=pltpu.CompilerParams(use_tc_tiling_on_sc=False),
        scratch_types=[
            pltpu.VMEM((QCH * CC,), jnp.int32),
            pltpu.VMEM((QCH * CC,), jnp.int32),
            pltpu.VMEM((QCH, CC), jnp.int32),
            [pltpu.VMEM((CC, DP), jnp.float32)] * NBUF,
            pltpu.VMEM((64, DP), jnp.float32),
            pltpu.VMEM_SHARED((ACCR, DP), jnp.float32),
            [pltpu.SemaphoreType.DMA] * NBUF,
            [pltpu.SemaphoreType.DMA] * NBUF,
        ],
    )
    def conv_kernel(table_hbm, src_hbm, dst_hbm, out_hbm, srcb, dstb, dst2d,
                    bufs, zbuf, acc, gsems, ssems):
        cid = lax.axis_index("c")
        sid = lax.axis_index("s")
        row0 = cid * HN

        def zfill(r, _):
            for k in range(DP // 16):
                zbuf[r, pl.ds(k * 16, 16)] = jnp.zeros((16,), jnp.float32)
            return 0

        lax.fori_loop(0, 64, zfill, 0)
        for i in range(OPT // 64):
            pltpu.sync_copy(zbuf, acc.at[pl.ds(sid * OPT + i * 64, 64)])

        @pl.when(sid == 0)
        def _():
            pltpu.sync_copy(zbuf.at[pl.ds(0, ACCR - HN)],
                            acc.at[pl.ds(HN, ACCR - HN)])

        def gstart(j, b):
            pltpu.async_copy(
                table_hbm.at[srcb.at[pl.ds(j * CC, CC)]], bufs[b], gsems[b])

        def gwait(j, b):
            pltpu.make_async_copy(
                table_hbm.at[srcb.at[pl.ds(j * CC, CC)]], bufs[b],
                gsems[b]).wait()

        def sstart(j, b):
            pltpu.async_copy(bufs[b], acc.at[dst2d.at[j]], ssems[b], add=True)

        def swait(j, b):
            pltpu.make_async_copy(bufs[b], acc.at[dst2d.at[j]],
                                  ssems[b]).wait()

        for q in range(NQ):
            base = (sid * NQ + q) * EPQ
            pltpu.sync_copy(src_hbm.at[pl.ds(base, EPQ)],
                            srcb.at[pl.ds(0, EPQ)])
            pltpu.sync_copy(dst_hbm.at[pl.ds(base, EPQ)],
                            dstb.at[pl.ds(0, EPQ)])
            _fill_pad(srcb, EPQ, QCH * CC - EPQ, NTOT, jnp.int32)
            _fill_pad(dstb, EPQ, QCH * CC - EPQ, NTOT, jnp.int32)

            def fix(ch, _):
                for k in range(CC // 16):
                    off = ch * CC + k * 16
                    s = srcb[pl.ds(off, 16)]
                    d = dstb[pl.ds(off, 16)]
                    srcb[pl.ds(off, 16)] = jnp.where(
                        s == d, jnp.full((16,), NTOT, jnp.int32), s)
                    dloc = d - row0
                    ooh = (dloc < 0) | (dloc >= HN)
                    dst2d[ch, pl.ds(k * 16, 16)] = jnp.where(
                        ooh, jnp.full((16,), JROW, jnp.int32), dloc)
                return 0

            lax.fori_loop(0, QCH, fix, 0)
            if q == 0:
                plsc.subcore_barrier()

            for b in range(NBUF):
                gstart(b, b)

            def loop(i, _):
                jj = NBUF * i
                for b in range(NBUF):
                    gwait(jj + b, b)
                    sstart(jj + b, b)
                    swait(jj + b, b)
                    gstart(jj + NBUF + b, b)
                return 0

            lax.fori_loop(0, QSCAT // NBUF, loop, 0)
            for b in range(NBUF):
                gwait(QSCAT + b, b)

        plsc.subcore_barrier()
        pltpu.sync_copy(acc.at[pl.ds(sid * OPT, OPT)], out_hbm.at[cid, sid])

    return conv_kernel


def _mlp_body(f_ref, w1_ref, b1_ref, w2_ref, b2_ref, o_ref):
    h = jnp.dot(f_ref[...], w1_ref[...], preferred_element_type=jnp.float32)
    h = h + b1_ref[...]
    h = jnp.where(h > 0, h, 0.01 * h)
    t = jnp.dot(h, w2_ref[...], preferred_element_type=jnp.float32)
    t = t + b2_ref[...]
    n = jnp.sqrt(jnp.sum(t * t, axis=1, keepdims=True))
    o_ref[...] = t / jnp.maximum(n, 1e-12)


def _pref_body(p_ref, o_ref):
    p = p_ref[...]
    n = jnp.sqrt(jnp.sum(p * p, axis=1, keepdims=True))
    o_ref[...] = p / jnp.maximum(n, 1e-12)


def _scale_body(degp_ref, x_ref, dinv_ref, xs_ref):
    deg = degp_ref[:, 0:1] + degp_ref[:, 1:2]
    dinv = jnp.where(deg > 0.0, lax.rsqrt(jnp.maximum(deg, 1e-12)), 0.0)
    dinv_ref[...] = dinv
    xs_ref[...] = x_ref[...] * dinv


def _mid_body(a_ref, dinv_ref, h_ref, ys_ref):
    dinv = dinv_ref[...]
    h = a_ref[...] * dinv
    h_ref[...] = h
    ys_ref[...] = h * dinv


def _final_body(x_ref, h_ref, b_ref, dinv_ref, o_ref):
    o_ref[...] = x_ref[...] + h_ref[...] + b_ref[...] * dinv_ref[...]


_deg_call = _make_deg()
_conv_call = _make_conv()


def kernel(features, preference, W1, b1, W2, b2, edge_index):
    temp_n = pl.pallas_call(
        _mlp_body,
        out_shape=jax.ShapeDtypeStruct((NUM_ITEM, D), jnp.float32),
    )(features, W1, b1.reshape(1, -1), W2, b2.reshape(1, -1))

    pref_n = pl.pallas_call(
        _pref_body,
        out_shape=jax.ShapeDtypeStruct((NUM_USER, D), jnp.float32),
    )(preference)

    x = jnp.concatenate([pref_n, temp_n], axis=0)
    x = jnp.pad(x, ((0, NPAD - NTOT), (0, 0)))

    e_src = edge_index[0]
    e_dst = edge_index[1]

    degp = _deg_call(e_src, e_dst)
    degp_t = degp.reshape(NC, NPAD).T

    dinv, xs = pl.pallas_call(
        _scale_body,
        out_shape=(
            jax.ShapeDtypeStruct((NPAD, 1), jnp.float32),
            jax.ShapeDtypeStruct((NPAD, DP), jnp.float32),
        ),
    )(degp_t, x)

    acc1 = _conv_call(xs, e_src, e_dst).reshape(NPAD, DP)

    h, ys = pl.pallas_call(
        _mid_body,
        out_shape=(
            jax.ShapeDtypeStruct((NPAD, DP), jnp.float32),
            jax.ShapeDtypeStruct((NPAD, DP), jnp.float32),
        ),
    )(acc1, dinv)

    acc2 = _conv_call(ys, e_src, e_dst).reshape(NPAD, DP)

    x_hat = pl.pallas_call(
        _final_body,
        out_shape=jax.ShapeDtypeStruct((NPAD, DP), jnp.float32),
    )(x, h, acc2, dinv)

    return x_hat[:NTOT]


# speedup vs baseline: 1.5209x; 1.0666x over previous
import functools

import jax
import jax.numpy as jnp
from jax import lax
from jax.experimental import pallas as pl
from jax.experimental.pallas import tpu as pltpu
from jax.experimental.pallas import tpu_sc as plsc

NUM_USER = 6000
NUM_ITEM = 4000
NTOT = NUM_USER + NUM_ITEM
NPAD = 10240
E = 640000
D = 64
DP = 64

NC = 2
NS = 16
NW = NC * NS
EPW = E // NW
C = 128
NCH = 160
NSCAT = 158
RPT = NPAD // NS
HN = NPAD // NC
JROW = HN
ACCR = HN + 8
OPT = HN // NS

_mesh = plsc.VectorSubcoreMesh(
    core_axis_name="c", subcore_axis_name="s", num_cores=NC, num_subcores=NS)


def _fill_pad(ref, start, count, value, dtype):
    vec = jnp.full((16,), value, dtype)

    def body(g, _):
        ref[pl.ds(start + g * 16, 16)] = vec
        return 0

    lax.fori_loop(0, count // 16, body, 0)


def _load_edges(src_hbm, dst_hbm, slot, srcb, dstb):
    base = slot * EPW
    pltpu.sync_copy(src_hbm.at[pl.ds(base, EPW)], srcb.at[pl.ds(0, EPW)])
    pltpu.sync_copy(dst_hbm.at[pl.ds(base, EPW)], dstb.at[pl.ds(0, EPW)])
    _fill_pad(srcb, EPW, NCH * C - EPW, NTOT, jnp.int32)
    _fill_pad(dstb, EPW, NCH * C - EPW, NTOT, jnp.int32)


def _make_deg():
    @functools.partial(
        pl.kernel,
        out_type=jax.ShapeDtypeStruct((NC, NS, RPT), jnp.float32),
        mesh=_mesh,
        compiler_params=pltpu.CompilerParams(use_tc_tiling_on_sc=False),
        scratch_types=[
            pltpu.VMEM((NCH * C,), jnp.int32),
            pltpu.VMEM((NCH * C,), jnp.int32),
            pltpu.VMEM((NCH, C), jnp.int32),
            pltpu.VMEM((NCH, C), jnp.float32),
            pltpu.VMEM((RPT,), jnp.float32),
            pltpu.VMEM_SHARED((NPAD,), jnp.float32),
            pltpu.SemaphoreType.DMA,
            pltpu.SemaphoreType.DMA,
        ],
    )
    def deg_kernel(src_hbm, dst_hbm, out_hbm, srcb, dstb, src2d, val2d, zdeg,
                   degacc, sem0, sem1):
        cid = lax.axis_index("c")
        sid = lax.axis_index("s")
        wid = sid * NC + cid

        _load_edges(src_hbm, dst_hbm, wid, srcb, dstb)

        def fix(ch, _):
            for k in range(8):
                off = ch * C + k * 16
                s = srcb[pl.ds(off, 16)]
                d = dstb[pl.ds(off, 16)]
                src2d[ch, pl.ds(k * 16, 16)] = s
                val2d[ch, pl.ds(k * 16, 16)] = jnp.where(
                    s == d, jnp.zeros((16,), jnp.float32),
                    jnp.ones((16,), jnp.float32))
            return 0

        lax.fori_loop(0, NCH, fix, 0)

        _fill_pad(zdeg, 0, RPT, 0.0, jnp.float32)
        pltpu.sync_copy(zdeg, degacc.at[pl.ds(sid * RPT, RPT)])
        plsc.subcore_barrier()

        def sstart(j, sem):
            pltpu.async_copy(val2d.at[j], degacc.at[src2d.at[j]], sem, add=True)

        def swait(j, sem):
            pltpu.make_async_copy(val2d.at[j], degacc.at[src2d.at[j]], sem).wait()

        def loop(i, _):
            jj = 2 * i
            sstart(jj, sem0)
            sstart(jj + 1, sem1)
            swait(jj, sem0)
            swait(jj + 1, sem1)
            return 0

        lax.fori_loop(0, NSCAT // 2, loop, 0)
        plsc.subcore_barrier()
        pltpu.sync_copy(degacc.at[pl.ds(sid * RPT, RPT)], out_hbm.at[cid, sid])

    return deg_kernel


NQ = 2
EPQ = E // (NS * NQ)
NBUF = 3
CC = 32
QSCAT = 627
QCH = QSCAT + 2


def _make_conv():
    @functools.partial(
        pl.kernel,
        out_type=jax.ShapeDtypeStruct((NC, NS, OPT, DP), jnp.float32),
        mesh=_mesh,
        compiler_params=pltpu.CompilerParams(use_tc_tiling_on_sc=False),
        scratch_types=[
            pltpu.VMEM((QCH * CC,), jnp.int32),
            pltpu.VMEM((QCH * CC,), jnp.int32),
            pltpu.VMEM((QCH, CC), jnp.int32),
            [pltpu.VMEM((CC, DP), jnp.float32)] * NBUF,
            pltpu.VMEM((64, DP), jnp.float32),
            pltpu.VMEM_SHARED((ACCR, DP), jnp.float32),
            [pltpu.SemaphoreType.DMA] * NBUF,
            [pltpu.SemaphoreType.DMA] * NBUF,
        ],
    )
    def conv_kernel(table_hbm, src_hbm, dst_hbm, out_hbm, srcb, dstb, dst2d,
                    bufs, zbuf, acc, gsems, ssems):
        cid = lax.axis_index("c")
        sid = lax.axis_index("s")
        row0 = cid * HN

        def zfill(r, _):
            for k in range(DP // 16):
                zbuf[r, pl.ds(k * 16, 16)] = jnp.zeros((16,), jnp.float32)
            return 0

        lax.fori_loop(0, 64, zfill, 0)
        for i in range(OPT // 64):
            pltpu.sync_copy(zbuf, acc.at[pl.ds(sid * OPT + i * 64, 64)])

        @pl.when(sid == 0)
        def _():
            pltpu.sync_copy(zbuf.at[pl.ds(0, ACCR - HN)],
                            acc.at[pl.ds(HN, ACCR - HN)])

        def gstart(j, b):
            pltpu.async_copy(
                table_hbm.at[srcb.at[pl.ds(j * CC, CC)]], bufs[b], gsems[b])

        def gwait(j, b):
            pltpu.make_async_copy(
                table_hbm.at[srcb.at[pl.ds(j * CC, CC)]], bufs[b],
                gsems[b]).wait()

        def sstart(j, b):
            pltpu.async_copy(bufs[b], acc.at[dst2d.at[j]], ssems[b], add=True)

        def swait(j, b):
            pltpu.make_async_copy(bufs[b], acc.at[dst2d.at[j]],
                                  ssems[b]).wait()

        for q in range(NQ):
            base = (sid * NQ + q) * EPQ
            pltpu.sync_copy(src_hbm.at[pl.ds(base, EPQ)],
                            srcb.at[pl.ds(0, EPQ)])
            pltpu.sync_copy(dst_hbm.at[pl.ds(base, EPQ)],
                            dstb.at[pl.ds(0, EPQ)])
            _fill_pad(srcb, EPQ, QCH * CC - EPQ, NTOT, jnp.int32)
            _fill_pad(dstb, EPQ, QCH * CC - EPQ, NTOT, jnp.int32)

            def fix(ch, _):
                for k in range(CC // 16):
                    off = ch * CC + k * 16
                    s = srcb[pl.ds(off, 16)]
                    d = dstb[pl.ds(off, 16)]
                    srcb[pl.ds(off, 16)] = jnp.where(
                        s == d, jnp.full((16,), NTOT, jnp.int32), s)
                    dloc = d - row0
                    ooh = (dloc < 0) | (dloc >= HN)
                    dst2d[ch, pl.ds(k * 16, 16)] = jnp.where(
                        ooh, jnp.full((16,), JROW, jnp.int32), dloc)
                return 0

            lax.fori_loop(0, QCH, fix, 0)
            if q == 0:
                plsc.subcore_barrier()

            gstart(0, 0)
            gstart(1, 1)
            for j in range(3):
                gwait(j, j % 3)
                sstart(j, j % 3)
                if j >= 1:
                    swait(j - 1, (j - 1) % 3)
                gstart(j + 2, (j + 2) % 3)

            def loop(i, _):
                jj = 3 * i
                for p in range(3):
                    j = jj + p
                    gwait(j, p)
                    sstart(j, p)
                    swait(j - 1, (p + 2) % 3)
                    gstart(j + 2, (p + 2) % 3)
                return 0

            lax.fori_loop(1, QSCAT // 3, loop, 0)
            swait(QSCAT - 1, (QSCAT - 1) % 3)
            gwait(QSCAT, QSCAT % 3)
            gwait(QSCAT + 1, (QSCAT + 1) % 3)

        plsc.subcore_barrier()
        pltpu.sync_copy(acc.at[pl.ds(sid * OPT, OPT)], out_hbm.at[cid, sid])

    return conv_kernel


def _mlp_body(f_ref, w1_ref, b1_ref, w2_ref, b2_ref, o_ref):
    h = jnp.dot(f_ref[...], w1_ref[...], preferred_element_type=jnp.float32)
    h = h + b1_ref[...]
    h = jnp.where(h > 0, h, 0.01 * h)
    t = jnp.dot(h, w2_ref[...], preferred_element_type=jnp.float32)
    t = t + b2_ref[...]
    n = jnp.sqrt(jnp.sum(t * t, axis=1, keepdims=True))
    o_ref[...] = t / jnp.maximum(n, 1e-12)


def _pref_body(p_ref, o_ref):
    p = p_ref[...]
    n = jnp.sqrt(jnp.sum(p * p, axis=1, keepdims=True))
    o_ref[...] = p / jnp.maximum(n, 1e-12)


def _scale_body(degp_ref, x_ref, dinv_ref, xs_ref):
    deg = degp_ref[:, 0:1] + degp_ref[:, 1:2]
    dinv = jnp.where(deg > 0.0, lax.rsqrt(jnp.maximum(deg, 1e-12)), 0.0)
    dinv_ref[...] = dinv
    xs_ref[...] = x_ref[...] * dinv


def _mid_body(a_ref, dinv_ref, h_ref, ys_ref):
    dinv = dinv_ref[...]
    h = a_ref[...] * dinv
    h_ref[...] = h
    ys_ref[...] = h * dinv


def _final_body(x_ref, h_ref, b_ref, dinv_ref, o_ref):
    o_ref[...] = x_ref[...] + h_ref[...] + b_ref[...] * dinv_ref[...]


_deg_call = _make_deg()
_conv_call = _make_conv()


def kernel(features, preference, W1, b1, W2, b2, edge_index):
    temp_n = pl.pallas_call(
        _mlp_body,
        out_shape=jax.ShapeDtypeStruct((NUM_ITEM, D), jnp.float32),
    )(features, W1, b1.reshape(1, -1), W2, b2.reshape(1, -1))

    pref_n = pl.pallas_call(
        _pref_body,
        out_shape=jax.ShapeDtypeStruct((NUM_USER, D), jnp.float32),
    )(preference)

    x = jnp.concatenate([pref_n, temp_n], axis=0)
    x = jnp.pad(x, ((0, NPAD - NTOT), (0, 0)))

    e_src = edge_index[0]
    e_dst = edge_index[1]

    degp = _deg_call(e_src, e_dst)
    degp_t = degp.reshape(NC, NPAD).T

    dinv, xs = pl.pallas_call(
        _scale_body,
        out_shape=(
            jax.ShapeDtypeStruct((NPAD, 1), jnp.float32),
            jax.ShapeDtypeStruct((NPAD, DP), jnp.float32),
        ),
    )(degp_t, x)

    acc1 = _conv_call(xs, e_src, e_dst).reshape(NPAD, DP)

    h, ys = pl.pallas_call(
        _mid_body,
        out_shape=(
            jax.ShapeDtypeStruct((NPAD, DP), jnp.float32),
            jax.ShapeDtypeStruct((NPAD, DP), jnp.float32),
        ),
    )(acc1, dinv)

    acc2 = _conv_call(ys, e_src, e_dst).reshape(NPAD, DP)

    x_hat = pl.pallas_call(
        _final_body,
        out_shape=jax.ShapeDtypeStruct((NPAD, DP), jnp.float32),
    )(x, h, acc2, dinv)

    return x_hat[:NTOT]


# speedup vs baseline: 1.5987x; 1.0511x over previous
import functools

import jax
import jax.numpy as jnp
from jax import lax
from jax.experimental import pallas as pl
from jax.experimental.pallas import tpu as pltpu
from jax.experimental.pallas import tpu_sc as plsc

NUM_USER = 6000
NUM_ITEM = 4000
NTOT = NUM_USER + NUM_ITEM
NPAD = 10240
E = 640000
D = 64
DP = 64

NC = 2
NS = 16
NW = NC * NS
EPW = E // NW
C = 128
NCH = 160
NSCAT = 158
RPT = NPAD // NS
HN = NPAD // NC
JROW = HN
ACCR = HN + 8
OPT = HN // NS

_mesh = plsc.VectorSubcoreMesh(
    core_axis_name="c", subcore_axis_name="s", num_cores=NC, num_subcores=NS)


def _fill_pad(ref, start, count, value, dtype):
    vec = jnp.full((16,), value, dtype)

    def body(g, _):
        ref[pl.ds(start + g * 16, 16)] = vec
        return 0

    lax.fori_loop(0, count // 16, body, 0)


def _load_edges(src_hbm, dst_hbm, slot, srcb, dstb):
    base = slot * EPW
    pltpu.sync_copy(src_hbm.at[pl.ds(base, EPW)], srcb.at[pl.ds(0, EPW)])
    pltpu.sync_copy(dst_hbm.at[pl.ds(base, EPW)], dstb.at[pl.ds(0, EPW)])
    _fill_pad(srcb, EPW, NCH * C - EPW, NTOT, jnp.int32)
    _fill_pad(dstb, EPW, NCH * C - EPW, NTOT, jnp.int32)


def _make_deg():
    @functools.partial(
        pl.kernel,
        out_type=jax.ShapeDtypeStruct((NC, NS, RPT), jnp.float32),
        mesh=_mesh,
        compiler_params=pltpu.CompilerParams(use_tc_tiling_on_sc=False),
        scratch_types=[
            pltpu.VMEM((NCH * C,), jnp.int32),
            pltpu.VMEM((NCH * C,), jnp.int32),
            pltpu.VMEM((NCH, C), jnp.int32),
            pltpu.VMEM((NCH, C), jnp.float32),
            pltpu.VMEM((RPT,), jnp.float32),
            pltpu.VMEM_SHARED((NPAD,), jnp.float32),
            pltpu.SemaphoreType.DMA,
            pltpu.SemaphoreType.DMA,
        ],
    )
    def deg_kernel(src_hbm, dst_hbm, out_hbm, srcb, dstb, src2d, val2d, zdeg,
                   degacc, sem0, sem1):
        cid = lax.axis_index("c")
        sid = lax.axis_index("s")
        wid = sid * NC + cid

        _load_edges(src_hbm, dst_hbm, wid, srcb, dstb)

        def fix(ch, _):
            for k in range(8):
                off = ch * C + k * 16
                s = srcb[pl.ds(off, 16)]
                d = dstb[pl.ds(off, 16)]
                src2d[ch, pl.ds(k * 16, 16)] = s
                val2d[ch, pl.ds(k * 16, 16)] = jnp.where(
                    s == d, jnp.zeros((16,), jnp.float32),
                    jnp.ones((16,), jnp.float32))
            return 0

        lax.fori_loop(0, NCH, fix, 0)

        _fill_pad(zdeg, 0, RPT, 0.0, jnp.float32)
        pltpu.sync_copy(zdeg, degacc.at[pl.ds(sid * RPT, RPT)])
        plsc.subcore_barrier()

        def sstart(j, sem):
            pltpu.async_copy(val2d.at[j], degacc.at[src2d.at[j]], sem, add=True)

        def swait(j, sem):
            pltpu.make_async_copy(val2d.at[j], degacc.at[src2d.at[j]], sem).wait()

        def loop(i, _):
            jj = 2 * i
            sstart(jj, sem0)
            sstart(jj + 1, sem1)
            swait(jj, sem0)
            swait(jj + 1, sem1)
            return 0

        lax.fori_loop(0, NSCAT // 2, loop, 0)
        plsc.subcore_barrier()
        pltpu.sync_copy(degacc.at[pl.ds(sid * RPT, RPT)], out_hbm.at[cid, sid])

    return deg_kernel


NQ = 2
EPQ = E // (NS * NQ)
NBUF = 3
CC = 48
QSCAT = 417
QCH = QSCAT + 2


def _make_conv():
    @functools.partial(
        pl.kernel,
        out_type=jax.ShapeDtypeStruct((NC, NS, OPT, DP), jnp.float32),
        mesh=_mesh,
        compiler_params=pltpu.CompilerParams(use_tc_tiling_on_sc=False),
        scratch_types=[
            pltpu.VMEM((QCH * CC,), jnp.int32),
            pltpu.VMEM((QCH * CC,), jnp.int32),
            pltpu.VMEM((QCH, CC), jnp.int32),
            [pltpu.VMEM((CC, DP), jnp.float32)] * NBUF,
            pltpu.VMEM((64, DP), jnp.float32),
            pltpu.VMEM_SHARED((ACCR, DP), jnp.float32),
            [pltpu.SemaphoreType.DMA] * NBUF,
            [pltpu.SemaphoreType.DMA] * NBUF,
        ],
    )
    def conv_kernel(table_hbm, src_hbm, dst_hbm, out_hbm, srcb, dstb, dst2d,
                    bufs, zbuf, acc, gsems, ssems):
        cid = lax.axis_index("c")
        sid = lax.axis_index("s")
        row0 = cid * HN

        def zfill(r, _):
            for k in range(DP // 16):
                zbuf[r, pl.ds(k * 16, 16)] = jnp.zeros((16,), jnp.float32)
            return 0

        lax.fori_loop(0, 64, zfill, 0)
        for i in range(OPT // 64):
            pltpu.sync_copy(zbuf, acc.at[pl.ds(sid * OPT + i * 64, 64)])

        @pl.when(sid == 0)
        def _():
            pltpu.sync_copy(zbuf.at[pl.ds(0, ACCR - HN)],
                            acc.at[pl.ds(HN, ACCR - HN)])

        def gstart(j, b):
            pltpu.async_copy(
                table_hbm.at[srcb.at[pl.ds(j * CC, CC)]], bufs[b], gsems[b])

        def gwait(j, b):
            pltpu.make_async_copy(
                table_hbm.at[srcb.at[pl.ds(j * CC, CC)]], bufs[b],
                gsems[b]).wait()

        def sstart(j, b):
            pltpu.async_copy(bufs[b], acc.at[dst2d.at[j]], ssems[b], add=True)

        def swait(j, b):
            pltpu.make_async_copy(bufs[b], acc.at[dst2d.at[j]],
                                  ssems[b]).wait()

        for q in range(NQ):
            base = (sid * NQ + q) * EPQ
            pltpu.sync_copy(src_hbm.at[pl.ds(base, EPQ)],
                            srcb.at[pl.ds(0, EPQ)])
            pltpu.sync_copy(dst_hbm.at[pl.ds(base, EPQ)],
                            dstb.at[pl.ds(0, EPQ)])
            _fill_pad(srcb, EPQ, QCH * CC - EPQ, NTOT, jnp.int32)
            _fill_pad(dstb, EPQ, QCH * CC - EPQ, NTOT, jnp.int32)

            def fix(ch, _):
                for k in range(CC // 16):
                    off = ch * CC + k * 16
                    s = srcb[pl.ds(off, 16)]
                    d = dstb[pl.ds(off, 16)]
                    srcb[pl.ds(off, 16)] = jnp.where(
                        s == d, jnp.full((16,), NTOT, jnp.int32), s)
                    dloc = d - row0
                    ooh = (dloc < 0) | (dloc >= HN)
                    dst2d[ch, pl.ds(k * 16, 16)] = jnp.where(
                        ooh, jnp.full((16,), JROW, jnp.int32), dloc)
                return 0

            lax.fori_loop(0, QCH, fix, 0)
            if q == 0:
                plsc.subcore_barrier()

            gstart(0, 0)
            gstart(1, 1)
            for j in range(3):
                gwait(j, j % 3)
                sstart(j, j % 3)
                if j >= 1:
                    swait(j - 1, (j - 1) % 3)
                gstart(j + 2, (j + 2) % 3)

            def loop(i, _):
                jj = 3 * i
                for p in range(3):
                    j = jj + p
                    gwait(j, p)
                    sstart(j, p)
                    swait(j - 1, (p + 2) % 3)
                    gstart(j + 2, (p + 2) % 3)
                return 0

            lax.fori_loop(1, QSCAT // 3, loop, 0)
            swait(QSCAT - 1, (QSCAT - 1) % 3)
            gwait(QSCAT, QSCAT % 3)
            gwait(QSCAT + 1, (QSCAT + 1) % 3)

        plsc.subcore_barrier()
        pltpu.sync_copy(acc.at[pl.ds(sid * OPT, OPT)], out_hbm.at[cid, sid])

    return conv_kernel


def _mlp_body(f_ref, w1_ref, b1_ref, w2_ref, b2_ref, o_ref):
    h = jnp.dot(f_ref[...], w1_ref[...], preferred_element_type=jnp.float32)
    h = h + b1_ref[...]
    h = jnp.where(h > 0, h, 0.01 * h)
    t = jnp.dot(h, w2_ref[...], preferred_element_type=jnp.float32)
    t = t + b2_ref[...]
    n = jnp.sqrt(jnp.sum(t * t, axis=1, keepdims=True))
    o_ref[...] = t / jnp.maximum(n, 1e-12)


def _pref_body(p_ref, o_ref):
    p = p_ref[...]
    n = jnp.sqrt(jnp.sum(p * p, axis=1, keepdims=True))
    o_ref[...] = p / jnp.maximum(n, 1e-12)


def _scale_body(degp_ref, x_ref, dinv_ref, xs_ref):
    deg = degp_ref[:, 0:1] + degp_ref[:, 1:2]
    dinv = jnp.where(deg > 0.0, lax.rsqrt(jnp.maximum(deg, 1e-12)), 0.0)
    dinv_ref[...] = dinv
    xs_ref[...] = x_ref[...] * dinv


def _mid_body(a_ref, dinv_ref, h_ref, ys_ref):
    dinv = dinv_ref[...]
    h = a_ref[...] * dinv
    h_ref[...] = h
    ys_ref[...] = h * dinv


def _final_body(x_ref, h_ref, b_ref, dinv_ref, o_ref):
    o_ref[...] = x_ref[...] + h_ref[...] + b_ref[...] * dinv_ref[...]


_deg_call = _make_deg()
_conv_call = _make_conv()


def kernel(features, preference, W1, b1, W2, b2, edge_index):
    temp_n = pl.pallas_call(
        _mlp_body,
        out_shape=jax.ShapeDtypeStruct((NUM_ITEM, D), jnp.float32),
    )(features, W1, b1.reshape(1, -1), W2, b2.reshape(1, -1))

    pref_n = pl.pallas_call(
        _pref_body,
        out_shape=jax.ShapeDtypeStruct((NUM_USER, D), jnp.float32),
    )(preference)

    x = jnp.concatenate([pref_n, temp_n], axis=0)
    x = jnp.pad(x, ((0, NPAD - NTOT), (0, 0)))

    e_src = edge_index[0]
    e_dst = edge_index[1]

    degp = _deg_call(e_src, e_dst)
    degp_t = degp.reshape(NC, NPAD).T

    dinv, xs = pl.pallas_call(
        _scale_body,
        out_shape=(
            jax.ShapeDtypeStruct((NPAD, 1), jnp.float32),
            jax.ShapeDtypeStruct((NPAD, DP), jnp.float32),
        ),
    )(degp_t, x)

    acc1 = _conv_call(xs, e_src, e_dst).reshape(NPAD, DP)

    h, ys = pl.pallas_call(
        _mid_body,
        out_shape=(
            jax.ShapeDtypeStruct((NPAD, DP), jnp.float32),
            jax.ShapeDtypeStruct((NPAD, DP), jnp.float32),
        ),
    )(acc1, dinv)

    acc2 = _conv_call(ys, e_src, e_dst).reshape(NPAD, DP)

    x_hat = pl.pallas_call(
        _final_body,
        out_shape=jax.ShapeDtypeStruct((NPAD, DP), jnp.float32),
    )(x, h, acc2, dinv)

    return x_hat[:NTOT]
